# baseline (speedup 1.0000x reference)
import jax
import jax.numpy as jnp
from jax.experimental import pallas as pl


def _agg_body(self_ref, emb_ref, mask_ref, na_ref, nb_ref, w_ref, b_ref,
              out_ref):
    nva = na_ref[:, 0]
    nvb = nb_ref[:, 0]
    h = nva.shape[1]
    m = mask_ref[...] * (1.0 / (2 * h))
    mean = (jnp.sum(nva * m[:, :h, None], axis=1)
            + jnp.sum(nvb * m[:, h:, None], axis=1))
    w = w_ref[...]
    d = mean.shape[1]
    acc = jnp.dot(self_ref[...], w[0:d], preferred_element_type=jnp.float32)
    acc = acc + jnp.dot(mean, w[d:2 * d], preferred_element_type=jnp.float32)
    acc = acc + jnp.dot(emb_ref[...], w[2 * d:3 * d],
                        preferred_element_type=jnp.float32)
    out_ref[...] = jnp.maximum(acc + b_ref[...], 0.0)


def kernel(self_vectors, neighbor_vectors, masks, node_emb, W, b):
    B_, _, H_, D_ = self_vectors.shape
    N_ = neighbor_vectors.shape[2]
    O_ = W.shape[1]
    BH = B_ * H_
    sv = self_vectors.reshape(BH, D_)
    nv4 = neighbor_vectors.reshape(BH, 2, N_ // 2, D_)
    mk = masks.reshape(BH, N_)
    ne = node_emb.reshape(BH, D_)
    b2 = b.reshape(1, O_)

    R = 2048
    grid = (BH // R,)
    out = pl.pallas_call(
        _agg_body,
        grid=grid,
        in_specs=[
            pl.BlockSpec((R, D_), lambda i: (i, 0)),
            pl.BlockSpec((R, D_), lambda i: (i, 0)),
            pl.BlockSpec((R, N_), lambda i: (i, 0)),
            pl.BlockSpec((R, 1, N_ // 2, D_), lambda i: (i, 0, 0, 0)),
            pl.BlockSpec((R, 1, N_ // 2, D_), lambda i: (i, 1, 0, 0)),
            pl.BlockSpec((3 * D_, O_), lambda i: (0, 0)),
            pl.BlockSpec((1, O_), lambda i: (0, 0)),
        ],
        out_specs=pl.BlockSpec((R, O_), lambda i: (i, 0)),
        out_shape=jax.ShapeDtypeStruct((BH, O_), jnp.float32),
    )(sv, ne, mk, nv4, nv4, W, b2)
    return out.reshape(B_, 1, H_, O_)
